# Initial kernel scaffold; baseline (speedup 1.0000x reference)
import jax
import jax.numpy as jnp
from jax.experimental import pallas as pl


def kernel(user_embed, item_embed, adj_values, adj_indices, users, pos_items):
    raise NotImplementedError("write your pallas kernel here")


# speedup vs baseline: 8.2519x; 8.2519x over previous
import functools

import jax
import jax.numpy as jnp
from jax import lax
from jax.experimental import pallas as pl
from jax.experimental.pallas import tpu as pltpu
from jax.experimental.pallas import tpu_sc as plsc

N_USERS = 50000
N_TOTAL = 100000
EMB = 32
NNZ = 1600000
BATCH = 4096
N_HOPS = 3

NC = 2
NS = 16
L = 16

HALF = N_TOTAL // NC
EDGES_PER_TILE = NNZ // NS
E_CHUNK = 400
N_CHUNKS = EDGES_PER_TILE // E_CHUNK
GROUPS = E_CHUNK // L
ROWS_PER_TILE = 3128
ACC_ROWS = ROWS_PER_TILE * NS
LAST_ROWS = HALF - 15 * ROWS_PER_TILE
ZROWS = 136

B_PER_W = BATCH // (NC * NS)

_mesh = plsc.VectorSubcoreMesh(core_axis_name="c", subcore_axis_name="s")


_GATHER_DNUMS = lax.GatherDimensionNumbers(
    offset_dims=(), collapsed_slice_dims=(0,), start_index_map=(0,))


def _bcast(vec, lane):
    idx = jnp.full((L, 1), lane, jnp.int32)
    return lax.gather(vec, idx, _GATHER_DNUMS, (1,),
                      mode=lax.GatherScatterMode.PROMISE_IN_BOUNDS)


def _hop_body(agg, rows_h, cols_h, vals_h, out_h,
              accum, colv, rowv, valv, gath, zbuf):
    c = lax.axis_index("c")
    s = lax.axis_index("s")

    zero = jnp.zeros((L,), jnp.float32)

    def _zb(g, _):
        zbuf[g, pl.ds(0, L)] = zero
        zbuf[g, pl.ds(L, L)] = zero
        return 0

    lax.fori_loop(0, ZROWS, _zb, 0)
    for k in range(ROWS_PER_TILE // ZROWS):
        pltpu.sync_copy(zbuf, accum.at[pl.ds(s * ROWS_PER_TILE + k * ZROWS, ZROWS)])
    plsc.subcore_barrier()

    def _chunk(k, _):
        eb = s * EDGES_PER_TILE + k * E_CHUNK
        pltpu.sync_copy(cols_h.at[pl.ds(eb, E_CHUNK)], colv)
        pltpu.sync_copy(rows_h.at[pl.ds(eb, E_CHUNK)], rowv)
        pltpu.sync_copy(vals_h.at[pl.ds(eb, E_CHUNK)], valv)
        pltpu.sync_copy(agg.at[colv], gath)

        def _grp(g, _):
            r16 = rowv[pl.ds(g * L, L)]
            v16 = valv[pl.ds(g * L, L)]
            upper01 = 1 + ((r16 - HALF) >> 31)
            idx = r16 - upper01 * HALF
            keep = 1 - (upper01 ^ c)
            v16 = v16 * keep.astype(jnp.float32)
            rowv[pl.ds(g * L, L)] = idx
            e0 = g * L
            for e in range(L):
                w = _bcast(v16, e)
                gath[e0 + e, pl.ds(0, L)] = gath[e0 + e, pl.ds(0, L)] * w
                gath[e0 + e, pl.ds(L, L)] = gath[e0 + e, pl.ds(L, L)] * w
            return 0

        lax.fori_loop(0, GROUPS, _grp, 0)
        pltpu.sync_copy(gath, accum.at[rowv], add=True)
        return 0

    lax.fori_loop(0, N_CHUNKS, _chunk, 0)
    plsc.subcore_barrier()

    @pl.when(s < NS - 1)
    def _flush_full():
        pltpu.sync_copy(
            accum.at[pl.ds(s * ROWS_PER_TILE, ROWS_PER_TILE)],
            out_h.at[pl.ds(c * HALF + s * ROWS_PER_TILE, ROWS_PER_TILE)])

    @pl.when(s == NS - 1)
    def _flush_last():
        pltpu.sync_copy(
            accum.at[pl.ds((NS - 1) * ROWS_PER_TILE, LAST_ROWS)],
            out_h.at[pl.ds(c * HALF + (NS - 1) * ROWS_PER_TILE, LAST_ROWS)])


_hop = pl.kernel(
    _hop_body,
    out_type=jax.ShapeDtypeStruct((N_TOTAL, EMB), jnp.float32),
    mesh=_mesh,
    compiler_params=pltpu.CompilerParams(use_tc_tiling_on_sc=False),
    scratch_types=[
        pltpu.VMEM_SHARED((ACC_ROWS, EMB), jnp.float32),
        pltpu.VMEM((E_CHUNK,), jnp.int32),
        pltpu.VMEM((E_CHUNK,), jnp.int32),
        pltpu.VMEM((E_CHUNK,), jnp.float32),
        pltpu.VMEM((E_CHUNK, EMB), jnp.float32),
        pltpu.VMEM((ZROWS, EMB), jnp.float32),
    ],
)


def _final_body(e0, e1, e2, e3, users_h, pos_h, out_h,
                uidx, pidx, ub0, ub1, ub2, ub3, ib0, ib1, ib2, ib3, outv):
    c = lax.axis_index("c")
    s = lax.axis_index("s")
    wid = s * NC + c
    base = wid * B_PER_W

    pltpu.sync_copy(users_h.at[pl.ds(base, B_PER_W)], uidx)
    pltpu.sync_copy(pos_h.at[pl.ds(base, B_PER_W)], pidx)

    def _shift(g, _):
        pidx[pl.ds(g * L, L)] = pidx[pl.ds(g * L, L)] + N_USERS
        return 0

    lax.fori_loop(0, B_PER_W // L, _shift, 0)

    pltpu.sync_copy(e0.at[uidx], ub0)
    pltpu.sync_copy(e1.at[uidx], ub1)
    pltpu.sync_copy(e2.at[uidx], ub2)
    pltpu.sync_copy(e3.at[uidx], ub3)
    pltpu.sync_copy(e0.at[pidx], ib0)
    pltpu.sync_copy(e1.at[pidx], ib1)
    pltpu.sync_copy(e2.at[pidx], ib2)
    pltpu.sync_copy(e3.at[pidx], ib3)

    def _dot(b, _):
        u0 = (ub0[b, pl.ds(0, L)] + ub1[b, pl.ds(0, L)]
              + ub2[b, pl.ds(0, L)] + ub3[b, pl.ds(0, L)])
        u1 = (ub0[b, pl.ds(L, L)] + ub1[b, pl.ds(L, L)]
              + ub2[b, pl.ds(L, L)] + ub3[b, pl.ds(L, L)])
        i0 = (ib0[b, pl.ds(0, L)] + ib1[b, pl.ds(0, L)]
              + ib2[b, pl.ds(0, L)] + ib3[b, pl.ds(0, L)])
        i1 = (ib0[b, pl.ds(L, L)] + ib1[b, pl.ds(L, L)]
              + ib2[b, pl.ds(L, L)] + ib3[b, pl.ds(L, L)])
        p = (u0 * i0 + u1 * i1) * (1.0 / 16.0)
        csum = plsc.cumsum(p)
        lane = lax.broadcasted_iota(jnp.int32, (L,), 0)
        plsc.store_scatter(outv, [jnp.full((L,), b, jnp.int32)], csum,
                           mask=lane == L - 1)
        return 0

    lax.fori_loop(0, B_PER_W, _dot, 0)
    pltpu.sync_copy(outv, out_h.at[pl.ds(base, B_PER_W)])


_final = pl.kernel(
    _final_body,
    out_type=jax.ShapeDtypeStruct((BATCH,), jnp.float32),
    mesh=_mesh,
    compiler_params=pltpu.CompilerParams(
        use_tc_tiling_on_sc=False, needs_layout_passes=False),
    scratch_types=(
        [pltpu.VMEM((B_PER_W,), jnp.int32)] * 2
        + [pltpu.VMEM((B_PER_W, EMB), jnp.float32)] * 8
        + [pltpu.VMEM((B_PER_W,), jnp.float32)]
    ),
)


def kernel(user_embed, item_embed, adj_values, adj_indices, users, pos_items):
    all_embed = jnp.concatenate([user_embed, item_embed], axis=0)
    rows = adj_indices[0]
    cols = adj_indices[1]
    e1 = _hop(all_embed, rows, cols, adj_values)
    e2 = _hop(e1, rows, cols, adj_values)
    e3 = _hop(e2, rows, cols, adj_values)
    return _final(all_embed, e1, e2, e3, users, pos_items)


# speedup vs baseline: 15.6927x; 1.9017x over previous
import functools

import jax
import jax.numpy as jnp
from jax import lax
from jax.experimental import pallas as pl
from jax.experimental.pallas import tpu as pltpu
from jax.experimental.pallas import tpu_sc as plsc

N_USERS = 50000
N_TOTAL = 100000
EMB = 32
NNZ = 1600000
BATCH = 4096
N_HOPS = 3

NC = 2
NS = 16
L = 16

HALF = N_TOTAL // NC
EDGES_PER_TILE = NNZ // NS
E_CHUNK = 400
N_CHUNKS = EDGES_PER_TILE // E_CHUNK
GROUPS = E_CHUNK // L
ROWS_PER_TILE = 3128
ACC_ROWS = ROWS_PER_TILE * NS
LAST_ROWS = HALF - 15 * ROWS_PER_TILE
ZROWS = 136

B_PER_W = BATCH // (NC * NS)

_mesh = plsc.VectorSubcoreMesh(core_axis_name="c", subcore_axis_name="s")


_GATHER_DNUMS = lax.GatherDimensionNumbers(
    offset_dims=(), collapsed_slice_dims=(0,), start_index_map=(0,))


def _bcast(vec, lane):
    idx = jnp.full((L, 1), lane, jnp.int32)
    return lax.gather(vec, idx, _GATHER_DNUMS, (1,),
                      mode=lax.GatherScatterMode.PROMISE_IN_BOUNDS)


def _hop_body(agg, rows_h, cols_h, vals_h, out_h,
              accum, colv0, colv1, rowv0, rowv1, valv0, valv1,
              idxv0, idxv1, gath0, gath1,
              sem_st0, sem_st1, sem_g0, sem_g1, sem_sc0, sem_sc1):
    c = lax.axis_index("c")
    s = lax.axis_index("s")
    colv = (colv0, colv1)
    rowv = (rowv0, rowv1)
    valv = (valv0, valv1)
    idxv = (idxv0, idxv1)
    gath = (gath0, gath1)
    sem_st = (sem_st0, sem_st1)
    sem_g = (sem_g0, sem_g1)
    sem_sc = (sem_sc0, sem_sc1)

    zero = jnp.zeros((L,), jnp.float32)

    def _zb(g, _):
        gath0[g, pl.ds(0, L)] = zero
        gath0[g, pl.ds(L, L)] = zero
        return 0

    lax.fori_loop(0, E_CHUNK, _zb, 0)
    zoff = 0
    while zoff < ROWS_PER_TILE:
        zn = min(E_CHUNK, ROWS_PER_TILE - zoff)
        pltpu.sync_copy(gath0.at[pl.ds(0, zn)],
                        accum.at[pl.ds(s * ROWS_PER_TILE + zoff, zn)])
        zoff += zn
    plsc.subcore_barrier()

    ebase = s * EDGES_PER_TILE

    def _stage(k, b):
        eb = ebase + k * E_CHUNK
        pltpu.async_copy(cols_h.at[pl.ds(eb, E_CHUNK)], colv[b], sem_st[b])
        pltpu.async_copy(rows_h.at[pl.ds(eb, E_CHUNK)], rowv[b], sem_st[b])
        pltpu.async_copy(vals_h.at[pl.ds(eb, E_CHUNK)], valv[b], sem_st[b])

    def _stage_wait(k, b):
        eb = ebase + k * E_CHUNK
        pltpu.make_async_copy(cols_h.at[pl.ds(eb, E_CHUNK)], colv[b], sem_st[b]).wait()
        pltpu.make_async_copy(rows_h.at[pl.ds(eb, E_CHUNK)], rowv[b], sem_st[b]).wait()
        pltpu.make_async_copy(vals_h.at[pl.ds(eb, E_CHUNK)], valv[b], sem_st[b]).wait()

    _stage(0, 0)

    def _outer(g, _):
        for b in range(2):
            k = 2 * g + b

            @pl.when(g >= 1)
            def _wait_sc():
                pltpu.make_async_copy(gath[b], accum.at[idxv[b]], sem_sc[b]).wait()

            _stage_wait(k, b)
            pltpu.async_copy(agg.at[colv[b]], gath[b], sem_g[b])

            @pl.when(k + 1 < N_CHUNKS)
            def _prefetch():
                _stage(k + 1, 1 - b)

            pltpu.make_async_copy(agg.at[colv[b]], gath[b], sem_g[b]).wait()

            def _grp(gi, _):
                r16 = rowv[b][pl.ds(gi * L, L)]
                v16 = valv[b][pl.ds(gi * L, L)]
                upper01 = 1 + ((r16 - HALF) >> 31)
                idx = r16 - upper01 * HALF
                keep = 1 - (upper01 ^ c)
                v16 = v16 * keep.astype(jnp.float32)
                idxv[b][pl.ds(gi * L, L)] = idx
                e0 = gi * L
                for e in range(L):
                    w = _bcast(v16, e)
                    gath[b][e0 + e, pl.ds(0, L)] = gath[b][e0 + e, pl.ds(0, L)] * w
                    gath[b][e0 + e, pl.ds(L, L)] = gath[b][e0 + e, pl.ds(L, L)] * w
                return 0

            lax.fori_loop(0, GROUPS, _grp, 0)
            pltpu.async_copy(gath[b], accum.at[idxv[b]], sem_sc[b], add=True)
        return 0

    lax.fori_loop(0, N_CHUNKS // 2, _outer, 0)
    for b in range(2):
        pltpu.make_async_copy(gath[b], accum.at[idxv[b]], sem_sc[b]).wait()
    plsc.subcore_barrier()

    @pl.when(s < NS - 1)
    def _flush_full():
        pltpu.sync_copy(
            accum.at[pl.ds(s * ROWS_PER_TILE, ROWS_PER_TILE)],
            out_h.at[pl.ds(c * HALF + s * ROWS_PER_TILE, ROWS_PER_TILE)])

    @pl.when(s == NS - 1)
    def _flush_last():
        pltpu.sync_copy(
            accum.at[pl.ds((NS - 1) * ROWS_PER_TILE, LAST_ROWS)],
            out_h.at[pl.ds(c * HALF + (NS - 1) * ROWS_PER_TILE, LAST_ROWS)])


_hop = pl.kernel(
    _hop_body,
    out_type=jax.ShapeDtypeStruct((N_TOTAL, EMB), jnp.float32),
    mesh=_mesh,
    compiler_params=pltpu.CompilerParams(use_tc_tiling_on_sc=False),
    scratch_types=(
        [pltpu.VMEM_SHARED((ACC_ROWS, EMB), jnp.float32)]
        + [pltpu.VMEM((E_CHUNK,), jnp.int32)] * 4
        + [pltpu.VMEM((E_CHUNK,), jnp.float32)] * 2
        + [pltpu.VMEM((E_CHUNK,), jnp.int32)] * 2
        + [pltpu.VMEM((E_CHUNK, EMB), jnp.float32)] * 2
        + [pltpu.SemaphoreType.DMA] * 6
    ),
)


def _final_body(e0, e1, e2, e3, users_h, pos_h, out_h,
                uidx, pidx, ub0, ub1, ub2, ub3, ib0, ib1, ib2, ib3, outv):
    c = lax.axis_index("c")
    s = lax.axis_index("s")
    wid = s * NC + c
    base = wid * B_PER_W

    pltpu.sync_copy(users_h.at[pl.ds(base, B_PER_W)], uidx)
    pltpu.sync_copy(pos_h.at[pl.ds(base, B_PER_W)], pidx)

    def _shift(g, _):
        pidx[pl.ds(g * L, L)] = pidx[pl.ds(g * L, L)] + N_USERS
        return 0

    lax.fori_loop(0, B_PER_W // L, _shift, 0)

    pltpu.sync_copy(e0.at[uidx], ub0)
    pltpu.sync_copy(e1.at[uidx], ub1)
    pltpu.sync_copy(e2.at[uidx], ub2)
    pltpu.sync_copy(e3.at[uidx], ub3)
    pltpu.sync_copy(e0.at[pidx], ib0)
    pltpu.sync_copy(e1.at[pidx], ib1)
    pltpu.sync_copy(e2.at[pidx], ib2)
    pltpu.sync_copy(e3.at[pidx], ib3)

    def _dot(b, _):
        u0 = (ub0[b, pl.ds(0, L)] + ub1[b, pl.ds(0, L)]
              + ub2[b, pl.ds(0, L)] + ub3[b, pl.ds(0, L)])
        u1 = (ub0[b, pl.ds(L, L)] + ub1[b, pl.ds(L, L)]
              + ub2[b, pl.ds(L, L)] + ub3[b, pl.ds(L, L)])
        i0 = (ib0[b, pl.ds(0, L)] + ib1[b, pl.ds(0, L)]
              + ib2[b, pl.ds(0, L)] + ib3[b, pl.ds(0, L)])
        i1 = (ib0[b, pl.ds(L, L)] + ib1[b, pl.ds(L, L)]
              + ib2[b, pl.ds(L, L)] + ib3[b, pl.ds(L, L)])
        p = (u0 * i0 + u1 * i1) * (1.0 / 16.0)
        csum = plsc.cumsum(p)
        lane = lax.broadcasted_iota(jnp.int32, (L,), 0)
        plsc.store_scatter(outv, [jnp.full((L,), b, jnp.int32)], csum,
                           mask=lane == L - 1)
        return 0

    lax.fori_loop(0, B_PER_W, _dot, 0)
    pltpu.sync_copy(outv, out_h.at[pl.ds(base, B_PER_W)])


_final = pl.kernel(
    _final_body,
    out_type=jax.ShapeDtypeStruct((BATCH,), jnp.float32),
    mesh=_mesh,
    compiler_params=pltpu.CompilerParams(
        use_tc_tiling_on_sc=False, needs_layout_passes=False),
    scratch_types=(
        [pltpu.VMEM((B_PER_W,), jnp.int32)] * 2
        + [pltpu.VMEM((B_PER_W, EMB), jnp.float32)] * 8
        + [pltpu.VMEM((B_PER_W,), jnp.float32)]
    ),
)


def kernel(user_embed, item_embed, adj_values, adj_indices, users, pos_items):
    all_embed = jnp.concatenate([user_embed, item_embed], axis=0)
    rows = adj_indices[0]
    cols = adj_indices[1]
    e1 = _hop(all_embed, rows, cols, adj_values)
    e2 = _hop(e1, rows, cols, adj_values)
    e3 = _hop(e2, rows, cols, adj_values)
    return _final(all_embed, e1, e2, e3, users, pos_items)
